# baseline (speedup 1.0000x reference)
import functools

import jax
import jax.numpy as jnp
from jax import lax
from jax.experimental import pallas as pl
from jax.experimental.pallas import tpu as pltpu
from jax.experimental.pallas import tpu_sc as plsc

_POINTER_OFFSET = 10
_T2T_PADDED = (0, 2, 1, 50260, 50261, 50262, 50263, 50258, 50259, 50257,
               0, 0, 0, 0, 0, 0)

_BATCH, _DEC_LEN, _ENC_LEN, _DIM = 4, 2048, 2048, 1024
_NW = 32
_PER_W = _BATCH * _DEC_LEN // _NW
_W_PER_ROW = _DEC_LEN // _PER_W
_CHUNK = 16
_NBUF = 4
_INFLIGHT = 3
_NCHUNK = _PER_W // _CHUNK
_LANES = 16

_mesh = plsc.VectorSubcoreMesh(core_axis_name="c", subcore_axis_name="s",
                               num_cores=2, num_subcores=16)


@functools.partial(
    pl.kernel,
    out_type=[
        jax.ShapeDtypeStruct((_BATCH, _DEC_LEN), jnp.int32),
        jax.ShapeDtypeStruct((_BATCH * _DEC_LEN, _DIM), jnp.float32),
    ],
    mesh=_mesh,
    compiler_params=pltpu.CompilerParams(needs_layout_passes=False,
                                         skip_device_barrier=True),
    scratch_types=[
        pltpu.VMEM((_DEC_LEN,), jnp.int32),
        pltpu.VMEM((_ENC_LEN,), jnp.int32),
        pltpu.VMEM((_LANES,), jnp.int32),
        pltpu.VMEM((1, _PER_W), jnp.int32),
        pltpu.VMEM((_NBUF, _CHUNK, _DIM), jnp.float32),
        pltpu.SemaphoreType.DMA,
        pltpu.SemaphoreType.DMA,
    ],
)
def _pointer_head_sc(ids_hbm, enc_hbm, table_hbm, t2t_hbm,
                     ids_out_hbm, emb_out_hbm,
                     ids_v, enc_v, t2t_v, dec_v, rows_v, gsem, osem):
    wid = lax.axis_index("s") * 2 + lax.axis_index("c")
    batch_row = wid // _W_PER_ROW
    col = (wid % _W_PER_ROW) * _PER_W
    base = wid * _PER_W

    pltpu.sync_copy(ids_hbm.at[batch_row], ids_v)
    pltpu.sync_copy(enc_hbm.at[batch_row], enc_v)
    pltpu.sync_copy(t2t_hbm, t2t_v)

    for i in range(_PER_W // _LANES):
        ids = ids_v[pl.ds(col + i * _LANES, _LANES)]
        is_tag = ids < _POINTER_OFFSET
        tag_idx = jnp.where(is_tag, ids, 0)
        tag_tok = plsc.load_gather(t2t_v, [tag_idx])
        enc_idx = jnp.minimum(jnp.maximum(ids - _POINTER_OFFSET, 0),
                              _ENC_LEN - 1)
        word_tok = plsc.load_gather(enc_v, [enc_idx])
        dec_v[0, pl.ds(i * _LANES, _LANES)] = jnp.where(is_tag, tag_tok,
                                                        word_tok)

    pltpu.sync_copy(dec_v,
                    ids_out_hbm.at[pl.ds(batch_row, 1), pl.ds(col, _PER_W)])

    def gather(c):
        return pltpu.async_copy(
            table_hbm.at[dec_v.at[0, pl.ds(c * _CHUNK, _CHUNK)]],
            rows_v.at[c % _NBUF], gsem)

    gd = [None] * _NCHUNK
    od = [None] * _NCHUNK
    for c in range(_INFLIGHT):
        gd[c] = gather(c)
    for c in range(_NCHUNK):
        gd[c].wait()
        od[c] = pltpu.async_copy(
            rows_v.at[c % _NBUF],
            emb_out_hbm.at[pl.ds(base + c * _CHUNK, _CHUNK)], osem)
        nxt = c + _INFLIGHT
        if nxt < _NCHUNK:
            if nxt >= _NBUF:
                od[nxt - _NBUF].wait()
            gd[nxt] = gather(nxt)
    for c in range(max(0, _NCHUNK - _NBUF), _NCHUNK):
        od[c].wait()


def kernel(input_ids, encoder_input_ids, table):
    t2t = jnp.array(_T2T_PADDED, dtype=jnp.int32)
    dec, emb = _pointer_head_sc(input_ids, encoder_input_ids, table, t2t)
    return (dec, emb.reshape(_BATCH, _DEC_LEN, _DIM))


# speedup vs baseline: 1.0081x; 1.0081x over previous
import functools

import jax
import jax.numpy as jnp
from jax import lax
from jax.experimental import pallas as pl
from jax.experimental.pallas import tpu as pltpu
from jax.experimental.pallas import tpu_sc as plsc

_POINTER_OFFSET = 10
_T2T_PADDED = (0, 2, 1, 50260, 50261, 50262, 50263, 50258, 50259, 50257,
               0, 0, 0, 0, 0, 0)

_BATCH, _DEC_LEN, _ENC_LEN, _DIM = 4, 2048, 2048, 1024
_NW = 32
_PER_W = _BATCH * _DEC_LEN // _NW
_W_PER_ROW = _DEC_LEN // _PER_W
_CHUNK = 16
_NBUF = 4
_INFLIGHT = 3
_NCHUNK = _PER_W // _CHUNK
_LANES = 16

_mesh = plsc.VectorSubcoreMesh(core_axis_name="c", subcore_axis_name="s",
                               num_cores=2, num_subcores=16)


@functools.partial(
    pl.kernel,
    out_type=[
        jax.ShapeDtypeStruct((_BATCH, _DEC_LEN), jnp.int32),
        jax.ShapeDtypeStruct((_BATCH * _DEC_LEN, _DIM), jnp.float32),
    ],
    mesh=_mesh,
    compiler_params=pltpu.CompilerParams(needs_layout_passes=False),
    scratch_types=[
        pltpu.VMEM((_DEC_LEN,), jnp.int32),
        pltpu.VMEM((_ENC_LEN,), jnp.int32),
        pltpu.VMEM((_LANES,), jnp.int32),
        pltpu.VMEM((1, _PER_W), jnp.int32),
        pltpu.VMEM((_NBUF, _CHUNK, _DIM), jnp.float32),
        pltpu.SemaphoreType.DMA,
        pltpu.SemaphoreType.DMA,
    ],
)
def _pointer_head_sc(ids_hbm, enc_hbm, table_hbm, t2t_hbm,
                     ids_out_hbm, emb_out_hbm,
                     ids_v, enc_v, t2t_v, dec_v, rows_v, gsem, osem):
    wid = lax.axis_index("s") * 2 + lax.axis_index("c")
    batch_row = wid // _W_PER_ROW
    col = (wid % _W_PER_ROW) * _PER_W
    base = wid * _PER_W

    pltpu.sync_copy(ids_hbm.at[batch_row], ids_v)
    pltpu.sync_copy(enc_hbm.at[batch_row], enc_v)
    pltpu.sync_copy(t2t_hbm, t2t_v)

    def compute_chunk(i):
        ids = ids_v[pl.ds(col + i * _LANES, _LANES)]
        is_tag = ids < _POINTER_OFFSET
        tag_idx = jnp.where(is_tag, ids, 0)
        tag_tok = plsc.load_gather(t2t_v, [tag_idx])
        enc_idx = jnp.minimum(jnp.maximum(ids - _POINTER_OFFSET, 0),
                              _ENC_LEN - 1)
        word_tok = plsc.load_gather(enc_v, [enc_idx])
        dec_v[0, pl.ds(i * _LANES, _LANES)] = jnp.where(is_tag, tag_tok,
                                                        word_tok)

    def gather(c):
        return pltpu.async_copy(
            table_hbm.at[dec_v.at[0, pl.ds(c * _CHUNK, _CHUNK)]],
            rows_v.at[c % _NBUF], gsem)

    gd = [None] * _NCHUNK
    od = [None] * _NCHUNK
    for c in range(_INFLIGHT):
        compute_chunk(c)
        gd[c] = gather(c)
    for c in range(_INFLIGHT, _NCHUNK):
        compute_chunk(c)
    pltpu.sync_copy(dec_v,
                    ids_out_hbm.at[pl.ds(batch_row, 1), pl.ds(col, _PER_W)])
    for c in range(_NCHUNK):
        gd[c].wait()
        od[c] = pltpu.async_copy(
            rows_v.at[c % _NBUF],
            emb_out_hbm.at[pl.ds(base + c * _CHUNK, _CHUNK)], osem)
        nxt = c + _INFLIGHT
        if nxt < _NCHUNK:
            if nxt >= _NBUF:
                od[nxt - _NBUF].wait()
            gd[nxt] = gather(nxt)
    for c in range(max(0, _NCHUNK - _NBUF), _NCHUNK):
        od[c].wait()


def kernel(input_ids, encoder_input_ids, table):
    t2t = jnp.array(_T2T_PADDED, dtype=jnp.int32)
    dec, emb = _pointer_head_sc(input_ids, encoder_input_ids, table, t2t)
    return (dec, emb.reshape(_BATCH, _DEC_LEN, _DIM))


# speedup vs baseline: 1.0233x; 1.0151x over previous
import functools

import jax
import jax.numpy as jnp
from jax import lax
from jax.experimental import pallas as pl
from jax.experimental.pallas import tpu as pltpu
from jax.experimental.pallas import tpu_sc as plsc

_POINTER_OFFSET = 10

_BATCH, _DEC_LEN, _ENC_LEN, _DIM = 4, 2048, 2048, 1024
_NW = 32
_PER_W = _BATCH * _DEC_LEN // _NW
_W_PER_ROW = _DEC_LEN // _PER_W
_CHUNK = 16
_NBUF = 4
_INFLIGHT = 3
_NCHUNK = _PER_W // _CHUNK
_LANES = 16

_mesh = plsc.VectorSubcoreMesh(core_axis_name="c", subcore_axis_name="s",
                               num_cores=2, num_subcores=16)


@functools.partial(
    pl.kernel,
    out_type=[
        jax.ShapeDtypeStruct((_BATCH, _DEC_LEN), jnp.int32),
        jax.ShapeDtypeStruct((_BATCH * _DEC_LEN, _DIM), jnp.float32),
    ],
    mesh=_mesh,
    compiler_params=pltpu.CompilerParams(needs_layout_passes=False),
    scratch_types=[
        pltpu.VMEM((_DEC_LEN,), jnp.int32),
        pltpu.VMEM((_ENC_LEN,), jnp.int32),
        pltpu.VMEM((1, _PER_W), jnp.int32),
        pltpu.VMEM((_NBUF, _CHUNK, _DIM), jnp.float32),
        pltpu.SemaphoreType.DMA,
        pltpu.SemaphoreType.DMA,
    ],
)
def _pointer_head_sc(ids_hbm, enc_hbm, table_hbm,
                     ids_out_hbm, emb_out_hbm,
                     ids_v, enc_v, dec_v, rows_v, gsem, osem):
    wid = lax.axis_index("s") * 2 + lax.axis_index("c")
    batch_row = wid // _W_PER_ROW
    col = (wid % _W_PER_ROW) * _PER_W
    base = wid * _PER_W

    pltpu.sync_copy(ids_hbm.at[batch_row], ids_v)
    pltpu.sync_copy(enc_hbm.at[batch_row], enc_v)

    def compute_chunk(i):
        ids = ids_v[pl.ds(col + i * _LANES, _LANES)]
        is_tag = ids < _POINTER_OFFSET
        small = jnp.where(ids == 1, 2, jnp.where(ids == 2, 1, 0))
        tag_tok = jnp.where(
            ids < 3, small,
            jnp.where(ids < 7, ids + 50257,
                      jnp.where(ids < 9, ids + 50251, 50257)))
        enc_idx = jnp.minimum(jnp.maximum(ids - _POINTER_OFFSET, 0),
                              _ENC_LEN - 1)
        word_tok = plsc.load_gather(enc_v, [enc_idx])
        dec_v[0, pl.ds(i * _LANES, _LANES)] = jnp.where(is_tag, tag_tok,
                                                        word_tok)

    def gather(c):
        return pltpu.async_copy(
            table_hbm.at[dec_v.at[0, pl.ds(c * _CHUNK, _CHUNK)]],
            rows_v.at[c % _NBUF], gsem)

    gd = [None] * _NCHUNK
    od = [None] * _NCHUNK
    for c in range(_INFLIGHT):
        compute_chunk(c)
        gd[c] = gather(c)
    for c in range(_INFLIGHT, _NCHUNK):
        compute_chunk(c)
    pltpu.sync_copy(dec_v,
                    ids_out_hbm.at[pl.ds(batch_row, 1), pl.ds(col, _PER_W)])
    for c in range(_NCHUNK):
        gd[c].wait()
        od[c] = pltpu.async_copy(
            rows_v.at[c % _NBUF],
            emb_out_hbm.at[pl.ds(base + c * _CHUNK, _CHUNK)], osem)
        nxt = c + _INFLIGHT
        if nxt < _NCHUNK:
            if nxt >= _NBUF:
                od[nxt - _NBUF].wait()
            gd[nxt] = gather(nxt)
    for c in range(max(0, _NCHUNK - _NBUF), _NCHUNK):
        od[c].wait()


def kernel(input_ids, encoder_input_ids, table):
    dec, emb = _pointer_head_sc(input_ids, encoder_input_ids, table)
    return (dec, emb.reshape(_BATCH, _DEC_LEN, _DIM))
